# baseline (speedup 1.0000x reference)
import functools

import jax
import jax.numpy as jnp
from jax.experimental import pallas as pl
import jax.experimental.pallas.tpu as pltpu


_MM = (((1,), (1,)), ((), ()))


def _body(h_ref, emb_ref, gum_ref,
          soft_ref, hard_ref, logits_ref, ids_ref, ent_ref,
          vn_ref, s_ref):
    @pl.when(pl.program_id(0) == 0)
    def _normalize_codebook():
        v = emb_ref[...]
        inv = 1.0 / jnp.maximum(
            jnp.sqrt(jnp.sum(v * v, axis=1, keepdims=True)), 1e-12)
        vn_ref[...] = (v * inv).astype(jnp.bfloat16)

    tm = h_ref.shape[0]
    i = pl.program_id(0)
    slot = jax.lax.rem(i, 2)

    ch = 128
    for c in range(tm // ch):
        rows = pl.ds(c * ch, ch)
        lg = s_ref[pl.ds((1 - slot) * tm + c * ch, ch), :]
        logits_ref[rows, :] = lg
        z = lg + gum_ref[rows, :]
        m2 = jnp.max(z, axis=-1, keepdims=True)
        e2 = jnp.exp(z)
        s2 = jnp.sum(e2, axis=-1, keepdims=True)
        soft_ref[rows, :] = e2 * (1.0 / s2)
        hard_ref[rows, :] = (z == m2).astype(jnp.float32)

        e1 = jnp.exp(lg)
        s1 = jnp.sum(e1, axis=-1, keepdims=True)
        ent = jnp.log(s1) - jnp.sum(e1 * lg, axis=-1, keepdims=True) / s1
        ent_ref[rows] = ent[:, 0]
        ids_ref[rows] = jnp.argmax(lg, axis=-1)

    h = h_ref[...]
    inv_n = 1.0 / jnp.maximum(
        jnp.sqrt(jnp.sum(h * h, axis=1, keepdims=True)), 1e-12)
    s_ref[pl.ds(slot * tm, tm), :] = jax.lax.dot_general(
        h * inv_n, vn_ref[...], dimension_numbers=_MM,
        preferred_element_type=jnp.float32)


@functools.partial(jax.jit, static_argnames=("tile_m",))
def _fused(h2, embeddings, gum2, tile_m):
    M, D = h2.shape
    K = embeddings.shape[0]
    n_tiles = M // tile_m
    grid = (n_tiles + 1,)
    last = n_tiles - 1

    def in_idx(i):
        return (jnp.minimum(i, last), 0)

    def out_idx(i):
        return (jnp.maximum(i - 1, 0), 0)

    def out_idx1(i):
        return (jnp.maximum(i - 1, 0),)

    out = pl.pallas_call(
        _body,
        grid=grid,
        in_specs=[
            pl.BlockSpec((tile_m, D), in_idx),
            pl.BlockSpec((K, D), lambda i: (0, 0)),
            pl.BlockSpec((tile_m, K), out_idx),
        ],
        out_specs=[
            pl.BlockSpec((tile_m, K), out_idx),
            pl.BlockSpec((tile_m, K), out_idx),
            pl.BlockSpec((tile_m, K), out_idx),
            pl.BlockSpec((tile_m,), out_idx1),
            pl.BlockSpec((tile_m,), out_idx1),
        ],
        out_shape=[
            jax.ShapeDtypeStruct((M, K), jnp.float32),
            jax.ShapeDtypeStruct((M, K), jnp.float32),
            jax.ShapeDtypeStruct((M, K), jnp.float32),
            jax.ShapeDtypeStruct((M,), jnp.int32),
            jax.ShapeDtypeStruct((M,), jnp.float32),
        ],
        scratch_shapes=[
            pltpu.VMEM((K, D), jnp.bfloat16),
            pltpu.VMEM((2 * tile_m, K), jnp.float32),
        ],
        compiler_params=pltpu.CompilerParams(
            dimension_semantics=("arbitrary",),
        ),
    )(h2, embeddings, gum2)
    return out


def kernel(h, embeddings, gumbels):
    B, S, D = h.shape
    K = embeddings.shape[0]
    M = B * S
    tile_m = 512 if M % 512 == 0 else M
    soft, hard, logits, ids, ent = _fused(
        h.reshape(M, D), embeddings, gumbels.reshape(M, K), tile_m)
    return (soft.reshape(B, S, K), hard.reshape(B, S, K), ids.reshape(B, S),
            logits.reshape(B, S, K), ent.reshape(B, S))


# speedup vs baseline: 1.0688x; 1.0688x over previous
import functools

import jax
import jax.numpy as jnp
from jax.experimental import pallas as pl
import jax.experimental.pallas.tpu as pltpu


_MM = (((1,), (1,)), ((), ()))


def _body(h_ref, emb_ref, gum_ref,
          soft_ref, hard_ref, logits_ref, ids_ref, ent_ref,
          vn_ref, s_ref):
    @pl.when(pl.program_id(0) == 0)
    def _normalize_codebook():
        v = emb_ref[...]
        inv = 1.0 / jnp.maximum(
            jnp.sqrt(jnp.sum(v * v, axis=1, keepdims=True)), 1e-12)
        vn_ref[...] = (v * inv).astype(jnp.bfloat16)

    tm = h_ref.shape[0]
    i = pl.program_id(0)
    slot = jax.lax.rem(i, 2)

    lg = s_ref[pl.ds((1 - slot) * tm, tm), :]
    logits_ref[...] = lg
    z = lg + gum_ref[...]
    m2 = jnp.max(z, axis=-1, keepdims=True)
    e2 = jnp.exp(z)
    s2 = jnp.sum(e2, axis=-1, keepdims=True)
    soft_ref[...] = e2 * (1.0 / s2)
    hard_ref[...] = (z == m2).astype(jnp.float32)

    e1 = jnp.exp(lg)
    s1 = jnp.sum(e1, axis=-1, keepdims=True)
    ent = jnp.log(s1) - jnp.sum(e1 * lg, axis=-1, keepdims=True) / s1
    ent_ref[...] = ent[:, 0]
    ids_ref[...] = jnp.argmax(lg, axis=-1)

    h = h_ref[...]
    inv_n = 1.0 / jnp.maximum(
        jnp.sqrt(jnp.sum(h * h, axis=1, keepdims=True)), 1e-12)
    s_ref[pl.ds(slot * tm, tm), :] = jax.lax.dot_general(
        h * inv_n, vn_ref[...], dimension_numbers=_MM,
        preferred_element_type=jnp.float32)


@functools.partial(jax.jit, static_argnames=("tile_m",))
def _fused(h2, embeddings, gum2, tile_m):
    M, D = h2.shape
    K = embeddings.shape[0]
    n_tiles = M // tile_m
    grid = (n_tiles + 1,)
    last = n_tiles - 1

    def in_idx(i):
        return (jnp.minimum(i, last), 0)

    def out_idx(i):
        return (jnp.maximum(i - 1, 0), 0)

    def out_idx1(i):
        return (jnp.maximum(i - 1, 0),)

    out = pl.pallas_call(
        _body,
        grid=grid,
        in_specs=[
            pl.BlockSpec((tile_m, D), in_idx),
            pl.BlockSpec((K, D), lambda i: (0, 0)),
            pl.BlockSpec((tile_m, K), out_idx),
        ],
        out_specs=[
            pl.BlockSpec((tile_m, K), out_idx),
            pl.BlockSpec((tile_m, K), out_idx),
            pl.BlockSpec((tile_m, K), out_idx),
            pl.BlockSpec((tile_m,), out_idx1),
            pl.BlockSpec((tile_m,), out_idx1),
        ],
        out_shape=[
            jax.ShapeDtypeStruct((M, K), jnp.float32),
            jax.ShapeDtypeStruct((M, K), jnp.float32),
            jax.ShapeDtypeStruct((M, K), jnp.float32),
            jax.ShapeDtypeStruct((M,), jnp.int32),
            jax.ShapeDtypeStruct((M,), jnp.float32),
        ],
        scratch_shapes=[
            pltpu.VMEM((K, D), jnp.bfloat16),
            pltpu.VMEM((2 * tile_m, K), jnp.float32),
        ],
        compiler_params=pltpu.CompilerParams(
            dimension_semantics=("arbitrary",),
        ),
    )(h2, embeddings, gum2)
    return out


def kernel(h, embeddings, gumbels):
    B, S, D = h.shape
    K = embeddings.shape[0]
    M = B * S
    tile_m = 512 if M % 512 == 0 else M
    soft, hard, logits, ids, ent = _fused(
        h.reshape(M, D), embeddings, gumbels.reshape(M, K), tile_m)
    return (soft.reshape(B, S, K), hard.reshape(B, S, K), ids.reshape(B, S),
            logits.reshape(B, S, K), ent.reshape(B, S))
